# baseline (speedup 1.0000x reference)
import functools

import jax
import jax.numpy as jnp
from jax.experimental import pallas as pl
from jax.experimental.pallas import tpu as pltpu


def _fused_kernel(item_idx_ref, x_ref, w_ref, b_ih_ref, b_hh_ref,
                  item_hbm_ref, user_hbm_ref, w_out_ref, b_out_ref,
                  out_ref, ubuf_ref, ibuf_ref, dma_sem, item_sem,
                  *, seq_len):
    cp = pltpu.make_async_copy(user_hbm_ref, ubuf_ref, dma_sem)
    cp.start()
    ip = pltpu.make_async_copy(item_hbm_ref.at[item_idx_ref[0]],
                               ibuf_ref, item_sem)
    ip.start()

    wih_cat = jnp.concatenate([w_ref[0], w_ref[1], w_ref[2]], axis=1)
    whh_cat = jnp.concatenate([w_ref[3], w_ref[4], w_ref[5]], axis=1)
    b_cat = jnp.concatenate(
        [b_ih_ref[0] + b_hh_ref[0], b_ih_ref[1] + b_hh_ref[1],
         b_ih_ref[2]], axis=1)
    b_hn = b_hh_ref[2]
    H = w_ref.shape[2]

    xb = x_ref[...].reshape(
        x_ref.shape[0], x_ref.shape[2]).astype(jnp.bfloat16)
    xcat = (jnp.dot(xb, wih_cat, preferred_element_type=jnp.float32)
            + b_cat)

    h = jnp.zeros((1, H), jnp.float32)
    h_sum = jnp.zeros((1, H), jnp.float32)

    for t in range(seq_len):
        xt = xcat[t:t + 1, :]
        hh = jnp.dot(h.astype(jnp.bfloat16), whh_cat,
                     preferred_element_type=jnp.float32)
        rz = jax.nn.sigmoid(xt[:, :2 * H] + hh[:, :2 * H])
        r = rz[:, :H]
        z = rz[:, H:]
        n = jnp.tanh(xt[:, 2 * H:] + r * (hh[:, 2 * H:] + b_hn))
        h = n + z * (h - n)
        h_sum = h_sum + h

    mean_h = h_sum * (1.0 / float(seq_len))
    ip.wait()
    scale = ibuf_ref[...] * mean_h

    cp.wait()

    mul = ubuf_ref[...] * scale
    logits = (jnp.dot(mul, w_out_ref[...],
                      preferred_element_type=jnp.float32)
              + b_out_ref[...])
    m = jnp.max(logits, axis=-1, keepdims=True)
    e = jnp.exp(logits - m)
    out_ref[...] = e / jnp.sum(e, axis=-1, keepdims=True)


def kernel(item_table, user_table, w_ih, w_hh, b_ih, b_hh, w_out, b_out,
           item_id, user_ids, word_embeddings):
    seq_len, batch, emb_dim = word_embeddings.shape
    hidden = w_hh.shape[-1]
    rating_range = w_out.shape[-1]
    assert batch == 1 and hidden == emb_dim

    user_emb = user_table[jnp.asarray(user_ids)]
    num_users = user_emb.shape[0]

    w_all = jnp.concatenate([w_ih, w_hh], axis=0).astype(jnp.bfloat16)
    item_idx = jnp.reshape(item_id, (1,))
    item3 = item_table.reshape(item_table.shape[0], 1, emb_dim)

    kern = functools.partial(_fused_kernel, seq_len=seq_len)
    grid_spec = pltpu.PrefetchScalarGridSpec(
        num_scalar_prefetch=1,
        grid=(1,),
        in_specs=[
            pl.BlockSpec((seq_len, 1, emb_dim), lambda i, ii: (0, 0, 0)),
            pl.BlockSpec((6, emb_dim, hidden), lambda i, ii: (0, 0, 0)),
            pl.BlockSpec((3, 1, hidden), lambda i, ii: (0, 0, 0)),
            pl.BlockSpec((3, 1, hidden), lambda i, ii: (0, 0, 0)),
            pl.BlockSpec(memory_space=pl.ANY),
            pl.BlockSpec(memory_space=pl.ANY),
            pl.BlockSpec((hidden, rating_range), lambda i, ii: (0, 0)),
            pl.BlockSpec((1, rating_range), lambda i, ii: (0, 0)),
        ],
        out_specs=pl.BlockSpec((num_users, rating_range),
                               lambda i, ii: (0, 0)),
        scratch_shapes=[
            pltpu.VMEM((num_users, emb_dim), jnp.float32),
            pltpu.VMEM((1, emb_dim), jnp.float32),
            pltpu.SemaphoreType.DMA,
            pltpu.SemaphoreType.DMA,
        ],
    )
    return pl.pallas_call(
        kern,
        out_shape=jax.ShapeDtypeStruct((num_users, rating_range),
                                       jnp.float32),
        grid_spec=grid_spec,
        compiler_params=pltpu.CompilerParams(
            dimension_semantics=("arbitrary",)),
    )(item_idx, word_embeddings, w_all, b_ih, b_hh, item3, user_emb,
      w_out, b_out)


# speedup vs baseline: 1.2013x; 1.2013x over previous
import functools

import jax
import jax.numpy as jnp
from jax.experimental import pallas as pl
from jax.experimental.pallas import tpu as pltpu


def _fused_kernel(item_idx_ref, x_ref, w_ref, b_ih_ref, b_hh_ref,
                  item_hbm_ref, user_hbm_ref, w_out_ref, b_out_ref,
                  out_ref, ubuf_ref, ibuf_ref, dma_sem, item_sem,
                  *, seq_len):
    cp = pltpu.make_async_copy(user_hbm_ref, ubuf_ref, dma_sem)
    cp.start()
    ip = pltpu.make_async_copy(item_hbm_ref.at[item_idx_ref[0]],
                               ibuf_ref, item_sem)
    ip.start()

    wih_cat = jnp.concatenate([w_ref[0], w_ref[1], w_ref[2]], axis=1)
    whh_cat = jnp.concatenate([w_ref[3], w_ref[4], w_ref[5]], axis=1)
    b_cat = jnp.concatenate(
        [b_ih_ref[0] + b_hh_ref[0], b_ih_ref[1] + b_hh_ref[1],
         b_ih_ref[2]], axis=1)
    b_hn = b_hh_ref[2]
    H = w_ref.shape[2]

    xb = x_ref[...]
    xcat = (jnp.dot(xb, wih_cat, preferred_element_type=jnp.float32)
            + b_cat)

    h = jnp.zeros((1, H), jnp.float32)
    h_sum = jnp.zeros((1, H), jnp.float32)

    for t in range(seq_len):
        xt = xcat[t:t + 1, :]
        hh = jnp.dot(h.astype(jnp.bfloat16), whh_cat,
                     preferred_element_type=jnp.float32)
        rz = jax.nn.sigmoid(xt[:, :2 * H] + hh[:, :2 * H])
        r = rz[:, :H]
        z = rz[:, H:]
        n = jnp.tanh(xt[:, 2 * H:] + r * (hh[:, 2 * H:] + b_hn))
        h = n + z * (h - n)
        h_sum = h_sum + h

    mean_h = h_sum * (1.0 / float(seq_len))
    ip.wait()
    scale = ibuf_ref[...].reshape(1, mean_h.shape[1]) * mean_h

    cp.wait()

    mul = ubuf_ref[...] * scale
    logits = (jnp.dot(mul, w_out_ref[...],
                      preferred_element_type=jnp.float32)
              + b_out_ref[...])
    m = jnp.max(logits, axis=-1, keepdims=True)
    e = jnp.exp(logits - m)
    out_ref[...] = e / jnp.sum(e, axis=-1, keepdims=True)


def kernel(item_table, user_table, w_ih, w_hh, b_ih, b_hh, w_out, b_out,
           item_id, user_ids, word_embeddings):
    seq_len, batch, emb_dim = word_embeddings.shape
    hidden = w_hh.shape[-1]
    rating_range = w_out.shape[-1]
    assert batch == 1 and hidden == emb_dim

    user_emb = user_table[jnp.asarray(user_ids)]
    num_users = user_emb.shape[0]

    xb = word_embeddings.reshape(seq_len, emb_dim).astype(jnp.bfloat16)
    w_all = jnp.concatenate([w_ih, w_hh], axis=0).astype(jnp.bfloat16)
    item_idx = jnp.reshape(item_id, (1,))

    kern = functools.partial(_fused_kernel, seq_len=seq_len)
    grid_spec = pltpu.PrefetchScalarGridSpec(
        num_scalar_prefetch=1,
        grid=(1,),
        in_specs=[
            pl.BlockSpec((seq_len, emb_dim), lambda i, ii: (0, 0)),
            pl.BlockSpec((6, emb_dim, hidden), lambda i, ii: (0, 0, 0)),
            pl.BlockSpec((3, 1, hidden), lambda i, ii: (0, 0, 0)),
            pl.BlockSpec((3, 1, hidden), lambda i, ii: (0, 0, 0)),
            pl.BlockSpec(memory_space=pl.ANY),
            pl.BlockSpec(memory_space=pl.ANY),
            pl.BlockSpec((hidden, rating_range), lambda i, ii: (0, 0)),
            pl.BlockSpec((1, rating_range), lambda i, ii: (0, 0)),
        ],
        out_specs=pl.BlockSpec((num_users, rating_range),
                               lambda i, ii: (0, 0)),
        scratch_shapes=[
            pltpu.VMEM((num_users, emb_dim), jnp.float32),
            pltpu.VMEM((emb_dim,), jnp.float32),
            pltpu.SemaphoreType.DMA,
            pltpu.SemaphoreType.DMA,
        ],
    )
    return pl.pallas_call(
        kern,
        out_shape=jax.ShapeDtypeStruct((num_users, rating_range),
                                       jnp.float32),
        grid_spec=grid_spec,
        compiler_params=pltpu.CompilerParams(
            dimension_semantics=("arbitrary",)),
    )(item_idx, xb, w_all, b_ih, b_hh, item_table, user_emb,
      w_out, b_out)
